# baseline (speedup 1.0000x reference)
import functools

import jax
import jax.numpy as jnp
from jax import lax
from jax.experimental import pallas as pl
from jax.experimental.pallas import tpu as pltpu
from jax.experimental.pallas import tpu_sc as plsc

N = 10000
D = 128
E = 320000
NC = 2
NS = 16
CHUNK = 128
KB = 8
EPT = -(-E // (NS * CHUNK * KB)) * CHUNK * KB
EPAD = EPT * NS
TBLK = EPAD // (CHUNK * KB)
BPT = TBLK // NS
NPAD = ((N + NS * CHUNK - 1) // (NS * CHUNK)) * NS * CHUNK
RPT = NPAD // NS
RCH = RPT // CHUNK


def _fill(ref, val):
    v = jnp.full((16,), val, jnp.float32)

    def row(r, _):
        for j in range(D // 16):
            ref[r, pl.ds(j * 16, 16)] = v
        return 0

    lax.fori_loop(0, CHUNK, row, 0, unroll=False)


def _sc_layer_body(x_hbm, src_hbm3, dst_hbm3, sums_hbm, cnts_hbm,
                   sblk, dblk, rows0, rows1, acc, gsem0, gsem1, ssem):
    cid = lax.axis_index("c")
    sid = lax.axis_index("s")
    r0 = sid * RPT
    _fill(rows0, 0.0)

    def zero(i, _):
        pltpu.sync_copy(rows0, acc.at[pl.ds(r0 + i * CHUNK, CHUNK)])
        return 0

    lax.fori_loop(0, RCH, zero, 0, unroll=False)
    plsc.subcore_barrier()

    b0 = sid * BPT
    bufs = (rows0, rows1)
    sems = (gsem0, gsem1)

    @pl.when(cid == 0)
    def _():
        def block(b, _):
            pltpu.sync_copy(src_hbm3.at[b0 + b], sblk)
            pltpu.sync_copy(dst_hbm3.at[b0 + b], dblk)
            gathers = [None, None]
            gathers[0] = pltpu.async_copy(x_hbm.at[sblk.at[0]], rows0, gsem0)
            for j in range(KB):
                if j + 1 < KB:
                    gathers[(j + 1) % 2] = pltpu.async_copy(
                        x_hbm.at[sblk.at[j + 1]], bufs[(j + 1) % 2],
                        sems[(j + 1) % 2])
                gathers[j % 2].wait()
                pltpu.sync_copy(bufs[j % 2], acc.at[dblk.at[j]], add=True)
            return 0

        lax.fori_loop(0, BPT, block, 0, unroll=False)

    @pl.when(cid == 1)
    def _():
        _fill(rows0, 1.0)

        def block2(t, _):
            for blk, boff in ((sblk, 0), (dblk, 1)):
                b = b0 + t * 2 + boff
                pltpu.sync_copy(dst_hbm3.at[b], blk)
                copies = [
                    pltpu.async_copy(rows0, acc.at[blk.at[j]], ssem, add=True)
                    for j in range(KB)
                ]
                for cpy in copies:
                    cpy.wait()
            return 0

        lax.fori_loop(0, BPT // 2, block2, 0, unroll=False)

    plsc.subcore_barrier()

    def out0(i, _):
        r = r0 + i * CHUNK
        pltpu.sync_copy(acc.at[pl.ds(r, CHUNK)], sums_hbm.at[pl.ds(r, CHUNK)])
        return 0

    def out1(i, _):
        r = r0 + i * CHUNK
        pltpu.sync_copy(acc.at[pl.ds(r, CHUNK)], cnts_hbm.at[pl.ds(r, CHUNK)])
        return 0

    @pl.when(cid == 0)
    def _():
        lax.fori_loop(0, RCH, out0, 0, unroll=False)

    @pl.when(cid == 1)
    def _():
        lax.fori_loop(0, RCH, out1, 0, unroll=False)


_SC_MESH = plsc.VectorSubcoreMesh(core_axis_name="c", subcore_axis_name="s",
                                  num_cores=NC, num_subcores=NS)

_LAYER_OUT = [jax.ShapeDtypeStruct((NPAD, D), jnp.float32),
              jax.ShapeDtypeStruct((NPAD, D), jnp.float32)]
_LAYER_SCRATCH = [
    pltpu.VMEM((KB, CHUNK), jnp.int32),
    pltpu.VMEM((KB, CHUNK), jnp.int32),
    pltpu.VMEM((CHUNK, D), jnp.float32),
    pltpu.VMEM((CHUNK, D), jnp.float32),
    pltpu.VMEM_SHARED((NPAD, D), jnp.float32),
    pltpu.SemaphoreType.DMA,
    pltpu.SemaphoreType.DMA,
    pltpu.SemaphoreType.DMA,
]

_sc_layer = pl.kernel(
    _sc_layer_body,
    out_type=_LAYER_OUT,
    mesh=_SC_MESH,
    scratch_types=_LAYER_SCRATCH,
)


BR = 1000


def _tc_body(sums_ref, cnts_ref, x_ref, wl_ref, wr_ref, b_ref, o_ref, *, last):
    s = sums_ref[...]
    c = cnts_ref[:, 0]
    mean = s / jnp.maximum(c, 1.0)[:, None]
    out = jnp.dot(mean, wl_ref[...], preferred_element_type=jnp.float32)
    out = out + jnp.dot(x_ref[...], wr_ref[...], preferred_element_type=jnp.float32)
    out = out + b_ref[...]
    if not last:
        out = jax.nn.gelu(out)
    o_ref[...] = out


def _tc_combine(sums, cnts, x, wl, wr, b, last):
    return pl.pallas_call(
        functools.partial(_tc_body, last=last),
        grid=(N // BR,),
        in_specs=[
            pl.BlockSpec((BR, D), lambda i: (i, 0)),
            pl.BlockSpec((BR, D), lambda i: (i, 0)),
            pl.BlockSpec((BR, D), lambda i: (i, 0)),
            pl.BlockSpec((D, D), lambda i: (0, 0)),
            pl.BlockSpec((D, D), lambda i: (0, 0)),
            pl.BlockSpec((1, D), lambda i: (0, 0)),
        ],
        out_specs=pl.BlockSpec((BR, D), lambda i: (i, 0)),
        out_shape=jax.ShapeDtypeStruct((N, D), jnp.float32),
    )(sums, cnts, x, wl, wr, b)


def kernel(x, edge_index0, edge_index1, edge_index2,
           Wl0, Wr0, b0, Wl1, Wr1, b1, Wl2, Wr2, b2):
    eis = (edge_index0, edge_index1, edge_index2)
    params = ((Wl0, Wr0, b0), (Wl1, Wr1, b1), (Wl2, Wr2, b2))
    pad = EPAD - E
    srcs = [jnp.concatenate([ei[0], jnp.zeros((pad,), jnp.int32)])
            .reshape(TBLK, KB, CHUNK) for ei in eis]
    dsts = [jnp.concatenate([ei[1], jnp.full((pad,), N, jnp.int32)])
            .reshape(TBLK, KB, CHUNK) for ei in eis]
    for i in range(3):
        sums, cnts = _sc_layer(x, srcs[i], dsts[i])
        wl, wr, b = params[i]
        x = _tc_combine(sums, cnts, x, wl, wr, b.reshape(1, D), last=(i == 2))
    return x
